# baseline (speedup 1.0000x reference)
import functools

import jax
import jax.numpy as jnp
from jax import lax
from jax.experimental import pallas as pl
from jax.experimental.pallas import tpu as pltpu
from jax.experimental.pallas import tpu_sc as plsc

N_LAYERS = 6
HID = 64
NODES_PAD = 50176
GROUP_NODES = 16
DMAX = 64
N_GROUPS = NODES_PAD // GROUP_NODES
GROUPS_PER_TILE = N_GROUPS // 32
GSLOTS = GROUP_NODES * DMAX
GCAP = 256
NEG = -3.0e38

_MESH = plsc.VectorSubcoreMesh(core_axis_name="c", subcore_axis_name="s")
_DNUMS = lax.GatherDimensionNumbers(
    offset_dims=(), collapsed_slice_dims=(0,), start_index_map=(0,))
_IN_BOUNDS = lax.GatherScatterMode.PROMISE_IN_BOUNDS


def _bcast(vec, l):
    idx = jnp.full((16, 1), l, jnp.int32)
    return lax.gather(vec, idx, _DNUMS, (1,), mode=_IN_BOUNDS)


def _vsum(p):
    for k in (1, 2, 4, 8):
        idx = (lax.iota(jnp.int32, 16) ^ k).reshape(16, 1)
        p = p + lax.gather(p, idx, _DNUMS, (1,), mode=_IN_BOUNDS)
    return p


def _edge_body(do_act, xl_hbm, xr_hbm, srcpad_hbm, deg_hbm, bb_hbm,
               out_hbm, idx_v, rows_v, xr_v, stg_v, deg_v, bias_v, sem):
    wid = lax.axis_index("s") * 2 + lax.axis_index("c")
    pltpu.sync_copy(bb_hbm, bias_v)
    a0 = bias_v[pl.ds(0, 16)]
    a1 = bias_v[pl.ds(16, 16)]
    a2 = bias_v[pl.ds(32, 16)]
    a3 = bias_v[pl.ds(48, 16)]
    b0 = bias_v[pl.ds(64, 16)]
    b1 = bias_v[pl.ds(80, 16)]
    b2 = bias_v[pl.ds(96, 16)]
    b3 = bias_v[pl.ds(112, 16)]
    negv = jnp.full((16,), NEG)

    def group(g, _):
        gg = wid * GROUPS_PER_TILE + g
        v0 = gg * GROUP_NODES
        pltpu.sync_copy(srcpad_hbm.at[gg], idx_v)
        pltpu.sync_copy(deg_hbm.at[gg], deg_v)
        pltpu.sync_copy(xr_hbm.at[pl.ds(v0, GROUP_NODES)], xr_v)
        cps = [pltpu.async_copy(xl_hbm.at[idx_v.at[j]],
                                rows_v.at[pl.ds(j * 128, 128)], sem)
               for j in range(GSLOTS // 128)]
        for cp in cps:
            cp.wait()

        def node(l, _):
            deg_l = _bcast(deg_v[...], l)
            xr0 = xr_v[l, pl.ds(0, 16)]
            xr1 = xr_v[l, pl.ds(16, 16)]
            xr2 = xr_v[l, pl.ds(32, 16)]
            xr3 = xr_v[l, pl.ds(48, 16)]

            def edge(t, carry):
                m, s, c0, c1, c2, c3 = carry
                q = l * DMAX + t
                x0 = rows_v[q, pl.ds(0, 16)]
                x1 = rows_v[q, pl.ds(16, 16)]
                x2 = rows_v[q, pl.ds(32, 16)]
                x3 = rows_v[q, pl.ds(48, 16)]
                z0 = x0 + xr0
                z1 = x1 + xr1
                z2 = x2 + xr2
                z3 = x3 + xr3
                z0 = jnp.maximum(z0, 0.2 * z0)
                z1 = jnp.maximum(z1, 0.2 * z1)
                z2 = jnp.maximum(z2, 0.2 * z2)
                z3 = jnp.maximum(z3, 0.2 * z3)
                e = _vsum(a0 * z0 + a1 * z1 + a2 * z2 + a3 * z3)
                pen = jnp.minimum(deg_l - 1 - t, 0).astype(jnp.float32)
                e = e + pen * 3.0e38
                mn = jnp.maximum(m, e)
                f = jnp.exp(m - mn)
                w = jnp.exp(e - mn)
                return (mn, s * f + w,
                        c0 * f + w * x0, c1 * f + w * x1,
                        c2 * f + w * x2, c3 * f + w * x3)

            z16 = jnp.zeros((16,), jnp.float32)
            m, s, c0, c1, c2, c3 = lax.fori_loop(
                0, DMAX, edge, (negv, z16, z16, z16, z16, z16))
            inv = 1.0 / s
            o0 = c0 * inv + b0
            o1 = c1 * inv + b1
            o2 = c2 * inv + b2
            o3 = c3 * inv + b3
            if do_act:
                o0 = jnp.maximum(o0, 0.01 * o0)
                o1 = jnp.maximum(o1, 0.01 * o1)
                o2 = jnp.maximum(o2, 0.01 * o2)
                o3 = jnp.maximum(o3, 0.01 * o3)
            stg_v[l, pl.ds(0, 16)] = o0
            stg_v[l, pl.ds(16, 16)] = o1
            stg_v[l, pl.ds(32, 16)] = o2
            stg_v[l, pl.ds(48, 16)] = o3
            return 0

        lax.fori_loop(0, GROUP_NODES, node, 0)
        pltpu.sync_copy(stg_v, out_hbm.at[pl.ds(v0, GROUP_NODES)])
        return 0

    lax.fori_loop(0, GROUPS_PER_TILE, group, 0)


def _make_edge_kernel(do_act):
    return functools.partial(
        pl.kernel, functools.partial(_edge_body, do_act),
        mesh=_MESH,
        compiler_params=pltpu.CompilerParams(use_tc_tiling_on_sc=False),
        out_type=jax.ShapeDtypeStruct((NODES_PAD, HID), jnp.float32),
        scratch_types=[
            pltpu.VMEM((GSLOTS // 128, 128), jnp.int32),
            pltpu.VMEM((GSLOTS, HID), jnp.float32),
            pltpu.VMEM((GROUP_NODES, HID), jnp.float32),
            pltpu.VMEM((GROUP_NODES, HID), jnp.float32),
            pltpu.VMEM((GROUP_NODES,), jnp.int32),
            pltpu.VMEM((128,), jnp.float32),
            pltpu.SemaphoreType.DMA,
        ],
    )()


_KERNEL_CACHE = {}


def _edge_kernels():
    if 'edge' not in _KERNEL_CACHE:
        _KERNEL_CACHE['edge'] = (_make_edge_kernel(True),
                                 _make_edge_kernel(False))
    return _KERNEL_CACHE['edge']


def _proj_body(h_ref, w_ref, o_ref):
    o_ref[...] = jnp.dot(h_ref[...], w_ref[...],
                         preferred_element_type=jnp.float32)


def _proj(h, w, block_rows=1568):
    n, d = h.shape
    _, dout = w.shape
    return pl.pallas_call(
        _proj_body,
        grid=(n // block_rows,),
        in_specs=[
            pl.BlockSpec((block_rows, d), lambda i: (i, 0)),
            pl.BlockSpec((d, dout), lambda i: (0, 0)),
        ],
        out_specs=pl.BlockSpec((block_rows, dout), lambda i: (i, 0)),
        out_shape=jax.ShapeDtypeStruct((n, dout), jnp.float32),
    )(h, w)


def _pool_body(h_hbm, gidx_hbm, cnt_hbm, out_hbm, idx_v, rows_v, stg_v,
               cnt_v, sem):
    wid = lax.axis_index("s") * 2 + lax.axis_index("c")
    pltpu.sync_copy(cnt_hbm.at[wid], cnt_v)
    negv = jnp.full((16,), NEG)

    def graph(l, _):
        g = wid * 16 + l
        pltpu.sync_copy(gidx_hbm.at[g], idx_v)
        cps = [pltpu.async_copy(h_hbm.at[idx_v.at[j]],
                                rows_v.at[pl.ds(j * 128, 128)], sem)
               for j in range(GCAP // 128)]
        for cp in cps:
            cp.wait()
        n_g = _bcast(cnt_v[...], l)
        n_f = n_g.astype(jnp.float32)

        def row(t, carry):
            s0, s1, s2, s3, m0, m1, m2, m3 = carry
            x0 = rows_v[t, pl.ds(0, 16)]
            x1 = rows_v[t, pl.ds(16, 16)]
            x2 = rows_v[t, pl.ds(32, 16)]
            x3 = rows_v[t, pl.ds(48, 16)]
            vf = jnp.clip(n_g - t, 0, 1).astype(jnp.float32)
            off = (vf - 1.0) * 3.0e38
            return (s0 + vf * x0, s1 + vf * x1, s2 + vf * x2, s3 + vf * x3,
                    jnp.maximum(m0, x0 * vf + off),
                    jnp.maximum(m1, x1 * vf + off),
                    jnp.maximum(m2, x2 * vf + off),
                    jnp.maximum(m3, x3 * vf + off))

        z16 = jnp.zeros((16,), jnp.float32)
        s0, s1, s2, s3, m0, m1, m2, m3 = lax.fori_loop(
            0, GCAP, row, (z16, z16, z16, z16, negv, negv, negv, negv))
        inv = 1.0 / jnp.maximum(n_f, 1.0)
        stg_v[l, pl.ds(0, 16)] = s0 * inv
        stg_v[l, pl.ds(16, 16)] = s1 * inv
        stg_v[l, pl.ds(32, 16)] = s2 * inv
        stg_v[l, pl.ds(48, 16)] = s3 * inv
        stg_v[l, pl.ds(64, 16)] = m0
        stg_v[l, pl.ds(80, 16)] = m1
        stg_v[l, pl.ds(96, 16)] = m2
        stg_v[l, pl.ds(112, 16)] = m3
        return 0

    lax.fori_loop(0, 16, graph, 0)
    pltpu.sync_copy(stg_v, out_hbm.at[pl.ds(wid * 16, 16)])


def _pool_kernel():
    if 'pool' not in _KERNEL_CACHE:
        _KERNEL_CACHE['pool'] = functools.partial(
            pl.kernel, _pool_body,
            mesh=_MESH,
            compiler_params=pltpu.CompilerParams(use_tc_tiling_on_sc=False),
            out_type=jax.ShapeDtypeStruct((512, 2 * HID), jnp.float32),
            scratch_types=[
                pltpu.VMEM((GCAP // 128, 128), jnp.int32),
                pltpu.VMEM((GCAP, HID), jnp.float32),
                pltpu.VMEM((16, 2 * HID), jnp.float32),
                pltpu.VMEM((16,), jnp.int32),
                pltpu.SemaphoreType.DMA,
            ],
        )()
    return _KERNEL_CACHE['pool']


def _mlp_body(z_ref, w1_ref, b1_ref, w2_ref, b2_ref, o_ref):
    t = jnp.dot(z_ref[...], w1_ref[...], preferred_element_type=jnp.float32)
    t = t + b1_ref[...]
    t = jnp.maximum(t, 0.01 * t)
    o_ref[...] = (jnp.dot(t, w2_ref[...], preferred_element_type=jnp.float32)
                  + b2_ref[...])


def _mlp(z, w1, b1, w2, b2):
    b = z.shape[0]
    return pl.pallas_call(
        _mlp_body,
        out_shape=jax.ShapeDtypeStruct((b, 128), jnp.float32),
    )(z, w1, b1[None, :], w2, b2[None, :])


def kernel(x, edge_index, batch, protein, params):
    n = x.shape[0]
    nb = protein.shape[0]

    loops = jnp.arange(n, dtype=edge_index.dtype)
    src = jnp.concatenate([edge_index[0], loops]).astype(jnp.int32)
    dst = jnp.concatenate([edge_index[1], loops]).astype(jnp.int32)
    order = jnp.argsort(dst)
    src_s = src[order]
    dst_s = dst[order]
    e_total = src_s.shape[0]
    row_off = jnp.searchsorted(
        dst_s, jnp.arange(NODES_PAD + 1, dtype=jnp.int32),
        side='left').astype(jnp.int32)
    pos = jnp.arange(e_total, dtype=jnp.int32) - row_off[dst_s]
    slot = jnp.where(pos < DMAX, dst_s * DMAX + pos,
                     jnp.int32(NODES_PAD * DMAX))
    srcpad = jnp.zeros((NODES_PAD * DMAX,), jnp.int32).at[slot].set(
        src_s, mode='drop').reshape(N_GROUPS, GSLOTS // 128, 128)
    deg = jnp.minimum(row_off[1:] - row_off[:-1],
                      DMAX).reshape(N_GROUPS, GROUP_NODES)

    boff = jnp.searchsorted(
        batch, jnp.arange(nb + 1, dtype=jnp.int32),
        side='left').astype(jnp.int32)
    gcnt = jnp.minimum(boff[1:] - boff[:-1], GCAP).astype(jnp.int32)
    gpos = jnp.arange(n, dtype=jnp.int32) - boff[batch]
    gslot = jnp.where(gpos < GCAP, batch * GCAP + gpos,
                      jnp.int32(nb * GCAP))
    gidx = jnp.zeros((nb * GCAP,), jnp.int32).at[gslot].set(
        jnp.arange(n, dtype=jnp.int32), mode='drop').reshape(
            nb, GCAP // 128, 128)
    gcnt2 = gcnt.reshape(32, 16)

    emb_idx = x[:, 0].astype(jnp.int32)
    q = params['emb'][emb_idx]
    h = jnp.concatenate([x, q], axis=1)
    h = jnp.pad(h, ((0, NODES_PAD - n), (0, 3)))

    ek_act, ek_noact = _edge_kernels()
    for i, lp in enumerate(params['gat']):
        d_in = lp['Wl'].shape[0]
        w = jnp.concatenate([lp['Wl'], lp['Wr']], axis=1)
        if d_in % 8 != 0:
            w = jnp.pad(w, ((0, h.shape[1] - d_in), (0, 0)))
        xlr = _proj(h, w)
        xl = xlr[:, :HID]
        xr = xlr[:, HID:]
        bb = jnp.concatenate([lp['att'], lp['b']])
        ek = ek_act if i < N_LAYERS - 1 else ek_noact
        h = ek(xl, xr, srcpad, deg, bb)

    pooled = _pool_kernel()(h, gidx, gcnt2)
    z = jnp.concatenate([pooled[:, :HID], protein, pooled[:, HID:]], axis=1)
    zp = jnp.pad(z, ((0, 0), (0, 5)))
    w1 = jnp.pad(params['fcW'], ((0, 5), (0, 0)))
    w2p = jnp.pad(params['fc2W'], ((0, 0), (0, 127)))
    b2p = jnp.pad(params['fc2b'], ((0, 127)))
    out = _mlp(zp, w1, params['fcb'], w2p, b2p)
    return out[:, :1]


# speedup vs baseline: 1.0001x; 1.0001x over previous
import functools

import jax
import jax.numpy as jnp
from jax import lax
from jax.experimental import pallas as pl
from jax.experimental.pallas import tpu as pltpu
from jax.experimental.pallas import tpu_sc as plsc

N_LAYERS = 6
HID = 64
NODES_PAD = 50176
GROUP_NODES = 16
DMAX = 64
N_GROUPS = NODES_PAD // GROUP_NODES
GROUPS_PER_TILE = N_GROUPS // 32
GSLOTS = GROUP_NODES * DMAX
GCAP = 256
NEG = -3.0e38

_MESH = plsc.VectorSubcoreMesh(core_axis_name="c", subcore_axis_name="s")
_DNUMS = lax.GatherDimensionNumbers(
    offset_dims=(), collapsed_slice_dims=(0,), start_index_map=(0,))
_IN_BOUNDS = lax.GatherScatterMode.PROMISE_IN_BOUNDS


def _bcast(vec, l):
    idx = jnp.full((16, 1), l, jnp.int32)
    return lax.gather(vec, idx, _DNUMS, (1,), mode=_IN_BOUNDS)


def _vsum(p):
    for k in (1, 2, 4, 8):
        idx = (lax.iota(jnp.int32, 16) ^ k).reshape(16, 1)
        p = p + lax.gather(p, idx, _DNUMS, (1,), mode=_IN_BOUNDS)
    return p


def _edge_body(do_act, xl_hbm, xr_hbm, srcpad_hbm, deg_hbm, bb_hbm,
               out_hbm, idx_v, rows_v, xr_v, stg_v, deg_v, bias_v, sem):
    wid = lax.axis_index("s") * 2 + lax.axis_index("c")
    pltpu.sync_copy(bb_hbm, bias_v)
    a0 = bias_v[pl.ds(0, 16)]
    a1 = bias_v[pl.ds(16, 16)]
    a2 = bias_v[pl.ds(32, 16)]
    a3 = bias_v[pl.ds(48, 16)]
    b0 = bias_v[pl.ds(64, 16)]
    b1 = bias_v[pl.ds(80, 16)]
    b2 = bias_v[pl.ds(96, 16)]
    b3 = bias_v[pl.ds(112, 16)]
    negv = jnp.full((16,), NEG)

    def group(g, _):
        gg = wid * GROUPS_PER_TILE + g
        v0 = gg * GROUP_NODES
        pltpu.sync_copy(srcpad_hbm.at[gg], idx_v)
        pltpu.sync_copy(deg_hbm.at[gg], deg_v)
        pltpu.sync_copy(xr_hbm.at[pl.ds(v0, GROUP_NODES)], xr_v)
        cps = [pltpu.async_copy(xl_hbm.at[idx_v.at[j]],
                                rows_v.at[pl.ds(j * 128, 128)], sem)
               for j in range(GSLOTS // 128)]
        for cp in cps:
            cp.wait()

        def node(l, _):
            deg_l = _bcast(deg_v[...], l)
            xr0 = xr_v[l, pl.ds(0, 16)]
            xr1 = xr_v[l, pl.ds(16, 16)]
            xr2 = xr_v[l, pl.ds(32, 16)]
            xr3 = xr_v[l, pl.ds(48, 16)]

            def edge(t, carry):
                m, s, c0, c1, c2, c3 = carry
                q = l * DMAX + t
                x0 = rows_v[q, pl.ds(0, 16)]
                x1 = rows_v[q, pl.ds(16, 16)]
                x2 = rows_v[q, pl.ds(32, 16)]
                x3 = rows_v[q, pl.ds(48, 16)]
                z0 = x0 + xr0
                z1 = x1 + xr1
                z2 = x2 + xr2
                z3 = x3 + xr3
                z0 = jnp.maximum(z0, 0.2 * z0)
                z1 = jnp.maximum(z1, 0.2 * z1)
                z2 = jnp.maximum(z2, 0.2 * z2)
                z3 = jnp.maximum(z3, 0.2 * z3)
                e = _vsum(a0 * z0 + a1 * z1 + a2 * z2 + a3 * z3)
                pen = jnp.minimum(deg_l - 1 - t, 0).astype(jnp.float32)
                e = e + pen * 3.0e38
                mn = jnp.maximum(m, e)
                f = jnp.exp(m - mn)
                w = jnp.exp(e - mn)
                return (mn, s * f + w,
                        c0 * f + w * x0, c1 * f + w * x1,
                        c2 * f + w * x2, c3 * f + w * x3)

            z16 = jnp.zeros((16,), jnp.float32)
            m, s, c0, c1, c2, c3 = lax.fori_loop(
                0, DMAX, edge, (negv, z16, z16, z16, z16, z16),
                unroll=8)
            inv = 1.0 / s
            o0 = c0 * inv + b0
            o1 = c1 * inv + b1
            o2 = c2 * inv + b2
            o3 = c3 * inv + b3
            if do_act:
                o0 = jnp.maximum(o0, 0.01 * o0)
                o1 = jnp.maximum(o1, 0.01 * o1)
                o2 = jnp.maximum(o2, 0.01 * o2)
                o3 = jnp.maximum(o3, 0.01 * o3)
            stg_v[l, pl.ds(0, 16)] = o0
            stg_v[l, pl.ds(16, 16)] = o1
            stg_v[l, pl.ds(32, 16)] = o2
            stg_v[l, pl.ds(48, 16)] = o3
            return 0

        lax.fori_loop(0, GROUP_NODES, node, 0)
        pltpu.sync_copy(stg_v, out_hbm.at[pl.ds(v0, GROUP_NODES)])
        return 0

    lax.fori_loop(0, GROUPS_PER_TILE, group, 0)


def _make_edge_kernel(do_act):
    return functools.partial(
        pl.kernel, functools.partial(_edge_body, do_act),
        mesh=_MESH,
        compiler_params=pltpu.CompilerParams(use_tc_tiling_on_sc=False),
        out_type=jax.ShapeDtypeStruct((NODES_PAD, HID), jnp.float32),
        scratch_types=[
            pltpu.VMEM((GSLOTS // 128, 128), jnp.int32),
            pltpu.VMEM((GSLOTS, HID), jnp.float32),
            pltpu.VMEM((GROUP_NODES, HID), jnp.float32),
            pltpu.VMEM((GROUP_NODES, HID), jnp.float32),
            pltpu.VMEM((GROUP_NODES,), jnp.int32),
            pltpu.VMEM((128,), jnp.float32),
            pltpu.SemaphoreType.DMA,
        ],
    )()


_KERNEL_CACHE = {}


def _edge_kernels():
    if 'edge' not in _KERNEL_CACHE:
        _KERNEL_CACHE['edge'] = (_make_edge_kernel(True),
                                 _make_edge_kernel(False))
    return _KERNEL_CACHE['edge']


def _proj_body(h_ref, w_ref, o_ref):
    o_ref[...] = jnp.dot(h_ref[...], w_ref[...],
                         preferred_element_type=jnp.float32)


def _proj(h, w, block_rows=1568):
    n, d = h.shape
    _, dout = w.shape
    return pl.pallas_call(
        _proj_body,
        grid=(n // block_rows,),
        in_specs=[
            pl.BlockSpec((block_rows, d), lambda i: (i, 0)),
            pl.BlockSpec((d, dout), lambda i: (0, 0)),
        ],
        out_specs=pl.BlockSpec((block_rows, dout), lambda i: (i, 0)),
        out_shape=jax.ShapeDtypeStruct((n, dout), jnp.float32),
    )(h, w)


def _pool_body(h_hbm, gidx_hbm, cnt_hbm, out_hbm, idx_v, rows_v, stg_v,
               cnt_v, sem):
    wid = lax.axis_index("s") * 2 + lax.axis_index("c")
    pltpu.sync_copy(cnt_hbm.at[wid], cnt_v)
    negv = jnp.full((16,), NEG)

    def graph(l, _):
        g = wid * 16 + l
        pltpu.sync_copy(gidx_hbm.at[g], idx_v)
        cps = [pltpu.async_copy(h_hbm.at[idx_v.at[j]],
                                rows_v.at[pl.ds(j * 128, 128)], sem)
               for j in range(GCAP // 128)]
        for cp in cps:
            cp.wait()
        n_g = _bcast(cnt_v[...], l)
        n_f = n_g.astype(jnp.float32)

        def row(t, carry):
            s0, s1, s2, s3, m0, m1, m2, m3 = carry
            x0 = rows_v[t, pl.ds(0, 16)]
            x1 = rows_v[t, pl.ds(16, 16)]
            x2 = rows_v[t, pl.ds(32, 16)]
            x3 = rows_v[t, pl.ds(48, 16)]
            vf = jnp.clip(n_g - t, 0, 1).astype(jnp.float32)
            off = (vf - 1.0) * 3.0e38
            return (s0 + vf * x0, s1 + vf * x1, s2 + vf * x2, s3 + vf * x3,
                    jnp.maximum(m0, x0 * vf + off),
                    jnp.maximum(m1, x1 * vf + off),
                    jnp.maximum(m2, x2 * vf + off),
                    jnp.maximum(m3, x3 * vf + off))

        z16 = jnp.zeros((16,), jnp.float32)
        s0, s1, s2, s3, m0, m1, m2, m3 = lax.fori_loop(
            0, GCAP, row, (z16, z16, z16, z16, negv, negv, negv, negv),
            unroll=8)
        inv = 1.0 / jnp.maximum(n_f, 1.0)
        stg_v[l, pl.ds(0, 16)] = s0 * inv
        stg_v[l, pl.ds(16, 16)] = s1 * inv
        stg_v[l, pl.ds(32, 16)] = s2 * inv
        stg_v[l, pl.ds(48, 16)] = s3 * inv
        stg_v[l, pl.ds(64, 16)] = m0
        stg_v[l, pl.ds(80, 16)] = m1
        stg_v[l, pl.ds(96, 16)] = m2
        stg_v[l, pl.ds(112, 16)] = m3
        return 0

    lax.fori_loop(0, 16, graph, 0)
    pltpu.sync_copy(stg_v, out_hbm.at[pl.ds(wid * 16, 16)])


def _pool_kernel():
    if 'pool' not in _KERNEL_CACHE:
        _KERNEL_CACHE['pool'] = functools.partial(
            pl.kernel, _pool_body,
            mesh=_MESH,
            compiler_params=pltpu.CompilerParams(use_tc_tiling_on_sc=False),
            out_type=jax.ShapeDtypeStruct((512, 2 * HID), jnp.float32),
            scratch_types=[
                pltpu.VMEM((GCAP // 128, 128), jnp.int32),
                pltpu.VMEM((GCAP, HID), jnp.float32),
                pltpu.VMEM((16, 2 * HID), jnp.float32),
                pltpu.VMEM((16,), jnp.int32),
                pltpu.SemaphoreType.DMA,
            ],
        )()
    return _KERNEL_CACHE['pool']


def _mlp_body(z_ref, w1_ref, b1_ref, w2_ref, b2_ref, o_ref):
    t = jnp.dot(z_ref[...], w1_ref[...], preferred_element_type=jnp.float32)
    t = t + b1_ref[...]
    t = jnp.maximum(t, 0.01 * t)
    o_ref[...] = (jnp.dot(t, w2_ref[...], preferred_element_type=jnp.float32)
                  + b2_ref[...])


def _mlp(z, w1, b1, w2, b2):
    b = z.shape[0]
    return pl.pallas_call(
        _mlp_body,
        out_shape=jax.ShapeDtypeStruct((b, 128), jnp.float32),
    )(z, w1, b1[None, :], w2, b2[None, :])


def kernel(x, edge_index, batch, protein, params):
    n = x.shape[0]
    nb = protein.shape[0]

    loops = jnp.arange(n, dtype=edge_index.dtype)
    src = jnp.concatenate([edge_index[0], loops]).astype(jnp.int32)
    dst = jnp.concatenate([edge_index[1], loops]).astype(jnp.int32)
    order = jnp.argsort(dst)
    src_s = src[order]
    dst_s = dst[order]
    e_total = src_s.shape[0]
    row_off = jnp.searchsorted(
        dst_s, jnp.arange(NODES_PAD + 1, dtype=jnp.int32),
        side='left').astype(jnp.int32)
    pos = jnp.arange(e_total, dtype=jnp.int32) - row_off[dst_s]
    slot = jnp.where(pos < DMAX, dst_s * DMAX + pos,
                     jnp.int32(NODES_PAD * DMAX))
    srcpad = jnp.zeros((NODES_PAD * DMAX,), jnp.int32).at[slot].set(
        src_s, mode='drop').reshape(N_GROUPS, GSLOTS // 128, 128)
    deg = jnp.minimum(row_off[1:] - row_off[:-1],
                      DMAX).reshape(N_GROUPS, GROUP_NODES)

    boff = jnp.searchsorted(
        batch, jnp.arange(nb + 1, dtype=jnp.int32),
        side='left').astype(jnp.int32)
    gcnt = jnp.minimum(boff[1:] - boff[:-1], GCAP).astype(jnp.int32)
    gpos = jnp.arange(n, dtype=jnp.int32) - boff[batch]
    gslot = jnp.where(gpos < GCAP, batch * GCAP + gpos,
                      jnp.int32(nb * GCAP))
    gidx = jnp.zeros((nb * GCAP,), jnp.int32).at[gslot].set(
        jnp.arange(n, dtype=jnp.int32), mode='drop').reshape(
            nb, GCAP // 128, 128)
    gcnt2 = gcnt.reshape(32, 16)

    emb_idx = x[:, 0].astype(jnp.int32)
    q = params['emb'][emb_idx]
    h = jnp.concatenate([x, q], axis=1)
    h = jnp.pad(h, ((0, NODES_PAD - n), (0, 3)))

    ek_act, ek_noact = _edge_kernels()
    for i, lp in enumerate(params['gat']):
        d_in = lp['Wl'].shape[0]
        w = jnp.concatenate([lp['Wl'], lp['Wr']], axis=1)
        if d_in % 8 != 0:
            w = jnp.pad(w, ((0, h.shape[1] - d_in), (0, 0)))
        xlr = _proj(h, w)
        xl = xlr[:, :HID]
        xr = xlr[:, HID:]
        bb = jnp.concatenate([lp['att'], lp['b']])
        ek = ek_act if i < N_LAYERS - 1 else ek_noact
        h = ek(xl, xr, srcpad, deg, bb)

    pooled = _pool_kernel()(h, gidx, gcnt2)
    z = jnp.concatenate([pooled[:, :HID], protein, pooled[:, HID:]], axis=1)
    zp = jnp.pad(z, ((0, 0), (0, 5)))
    w1 = jnp.pad(params['fcW'], ((0, 5), (0, 0)))
    w2p = jnp.pad(params['fc2W'], ((0, 0), (0, 127)))
    b2p = jnp.pad(params['fc2b'], ((0, 127)))
    out = _mlp(zp, w1, params['fcb'], w2p, b2p)
    return out[:, :1]
